# baseline (speedup 1.0000x reference)
import functools
import math

import jax
import jax.numpy as jnp
from jax import lax
from jax.experimental import pallas as pl
from jax.experimental.pallas import tpu as pltpu
from jax.experimental.pallas import tpu_sc as plsc

_D = 64
_SCALE = math.sqrt(_D)
_IW = 100
_R0 = 4
_GPC = 8
_NBUF = 2


def _emb_body(idx_hbm, table_hbm, out_hbm, idx_v, buf0, buf1, g_sems, w_sems,
              *, nc, w_rows, n_chunks):
    wid = lax.axis_index("s") * nc + lax.axis_index("c")
    base = wid * w_rows
    bufs = (buf0, buf1)

    pltpu.sync_copy(idx_hbm.at[pl.ds(wid * w_rows * 2, w_rows * 2)], idx_v)

    def fire_gathers(s, g):
        for k in range(_GPC):
            pltpu.async_copy(
                table_hbm.at[idx_v.at[g * _GPC + k]],
                bufs[s].at[k // 2, pl.ds((k % 2) * _IW, _IW)],
                g_sems.at[s],
            )

    def wait_gathers(s):
        for k in range(_GPC):
            pltpu.make_async_copy(
                table_hbm.at[idx_v.at[0]],
                bufs[s].at[k // 2, pl.ds((k % 2) * _IW, _IW)],
                g_sems.at[s],
            ).wait()

    def fire_write(s, g):
        pltpu.async_copy(
            bufs[s], out_hbm.at[pl.ds(base + g * _R0, _R0)], w_sems.at[s])

    def wait_write(s):
        pltpu.make_async_copy(
            bufs[s], out_hbm.at[pl.ds(base, _R0)], w_sems.at[s]).wait()

    fire_gathers(0, 0)
    fire_gathers(1, 1)

    @pl.loop(0, n_chunks // 2 - 1)
    def _steady(g2):
        c0 = g2 * 2
        for s in range(_NBUF):
            wait_gathers(s)
            fire_write(s, c0 + s)
            wait_write(s)
            fire_gathers(s, c0 + s + 2)

    for s in range(_NBUF):
        wait_gathers(s)
        fire_write(s, n_chunks - 2 + s)
    for s in range(_NBUF):
        wait_write(s)


def kernel(x, table):
    b0, b1 = x.shape
    idx = x.reshape(b0 * b1 // _IW, _IW).astype(jnp.int32)
    tab = table * _SCALE

    info = plsc.get_sparse_core_info()
    nc, ns = info.num_cores, info.num_subcores
    nw = nc * ns
    w_rows = b0 // nw
    n_chunks = w_rows // _R0

    mesh = plsc.VectorSubcoreMesh(core_axis_name="c", subcore_axis_name="s")
    emb = pl.kernel(
        functools.partial(_emb_body, nc=nc, w_rows=w_rows, n_chunks=n_chunks),
        out_type=jax.ShapeDtypeStruct((b0, b1, _D), jnp.float32),
        mesh=mesh,
        compiler_params=pltpu.CompilerParams(use_tc_tiling_on_sc=False),
        scratch_types=[
            pltpu.VMEM((b0 * b1 // (nw * _IW), _IW), jnp.int32),
            pltpu.VMEM((_R0, b1, _D), jnp.float32),
            pltpu.VMEM((_R0, b1, _D), jnp.float32),
            pltpu.SemaphoreType.DMA((_NBUF,)),
            pltpu.SemaphoreType.DMA((_NBUF,)),
        ],
    )
    return emb(idx, tab)


# speedup vs baseline: 1.2531x; 1.2531x over previous
import functools
import math

import jax
import jax.numpy as jnp
from jax import lax
from jax.experimental import pallas as pl
from jax.experimental.pallas import tpu as pltpu
from jax.experimental.pallas import tpu_sc as plsc

_D = 64
_SCALE = math.sqrt(_D)
_IW = 40
_R0 = 4
_GPC = 20
_NBUF = 2


def _emb_body(idx_hbm, table_hbm, out_hbm, idx_v, buf0, buf1, g_sems, w_sems,
              *, nc, w_rows, b1, n_chunks):
    wid = lax.axis_index("s") * nc + lax.axis_index("c")
    base = wid * w_rows
    fbase = base * b1
    bufs = (buf0, buf1)

    pltpu.sync_copy(idx_hbm.at[pl.ds(fbase, w_rows * b1)], idx_v)

    def fire_gathers(s, g):
        for k in range(_GPC):
            pltpu.async_copy(
                table_hbm.at[idx_v.at[pl.ds((g * _GPC + k) * _IW, _IW)]],
                bufs[s].at[k // 5, pl.ds((k % 5) * _IW, _IW)],
                g_sems.at[s],
            )

    def wait_gathers(s):
        for k in range(_GPC):
            pltpu.make_async_copy(
                table_hbm.at[idx_v.at[pl.ds(0, _IW)]],
                bufs[s].at[k // 5, pl.ds((k % 5) * _IW, _IW)],
                g_sems.at[s],
            ).wait()

    def scale(s):
        @pl.loop(0, b1)
        def _rows(r):
            for d in range(_R0):
                for j in range(_D // 16):
                    sl = pl.ds(j * 16, 16)
                    bufs[s][d, r, sl] = bufs[s][d, r, sl] * _SCALE

    def fire_write(s, g):
        pltpu.async_copy(
            bufs[s], out_hbm.at[pl.ds(base + g * _R0, _R0)], w_sems.at[s])

    def wait_write(s):
        pltpu.make_async_copy(
            bufs[s], out_hbm.at[pl.ds(base, _R0)], w_sems.at[s]).wait()

    fire_gathers(0, 0)
    fire_gathers(1, 1)

    @pl.loop(0, n_chunks // 2 - 1)
    def _steady(g2):
        c0 = g2 * 2
        for s in range(_NBUF):
            wait_gathers(s)
            scale(s)
            fire_write(s, c0 + s)
            wait_write(s)
            fire_gathers(s, c0 + s + 2)

    for s in range(_NBUF):
        wait_gathers(s)
        scale(s)
        fire_write(s, n_chunks - 2 + s)
    for s in range(_NBUF):
        wait_write(s)


def kernel(x, table):
    b0, b1 = x.shape
    idx = x.reshape(b0 * b1).astype(jnp.int32)

    info = plsc.get_sparse_core_info()
    nc, ns = info.num_cores, info.num_subcores
    nw = nc * ns
    w_rows = b0 // nw
    n_chunks = w_rows // _R0

    mesh = plsc.VectorSubcoreMesh(core_axis_name="c", subcore_axis_name="s")
    emb = pl.kernel(
        functools.partial(_emb_body, nc=nc, w_rows=w_rows, b1=b1,
                          n_chunks=n_chunks),
        out_type=jax.ShapeDtypeStruct((b0, b1, _D), jnp.float32),
        mesh=mesh,
        compiler_params=pltpu.CompilerParams(
            use_tc_tiling_on_sc=False, needs_layout_passes=True),
        scratch_types=[
            pltpu.VMEM((b0 * b1 // nw,), jnp.int32),
            pltpu.VMEM((_R0, b1, _D), jnp.float32),
            pltpu.VMEM((_R0, b1, _D), jnp.float32),
            pltpu.SemaphoreType.DMA((_NBUF,)),
            pltpu.SemaphoreType.DMA((_NBUF,)),
        ],
    )
    return emb(idx, table)


# speedup vs baseline: 1.3172x; 1.0511x over previous
import functools
import math

import jax
import jax.numpy as jnp
from jax import lax
from jax.experimental import pallas as pl
from jax.experimental.pallas import tpu as pltpu
from jax.experimental.pallas import tpu_sc as plsc

_D = 64
_SCALE = math.sqrt(_D)
_IW = 40
_R0 = 4
_GPC = 20
_NBUF = 2


def _emb_body(idx_hbm, table_hbm, out_hbm, idx_v, buf0, buf1, g_sems, w_sems,
              *, nc, w_rows, b1, n_chunks):
    wid = lax.axis_index("s") * nc + lax.axis_index("c")
    base = wid * w_rows
    fbase = base * b1
    bufs = (buf0, buf1)

    pltpu.sync_copy(idx_hbm.at[pl.ds(fbase, w_rows * b1)], idx_v)

    def fire_gathers(s, g):
        for k in range(_GPC):
            pltpu.async_copy(
                table_hbm.at[idx_v.at[pl.ds((g * _GPC + k) * _IW, _IW)]],
                bufs[s].at[k // 5, pl.ds((k % 5) * _IW, _IW)],
                g_sems.at[s],
            )

    def wait_gathers(s):
        for k in range(_GPC):
            pltpu.make_async_copy(
                table_hbm.at[idx_v.at[pl.ds(0, _IW)]],
                bufs[s].at[k // 5, pl.ds((k % 5) * _IW, _IW)],
                g_sems.at[s],
            ).wait()

    def scale(s):
        @pl.loop(0, b1)
        def _rows(r):
            for d in range(_R0):
                for j in range(_D // 16):
                    sl = pl.ds(j * 16, 16)
                    bufs[s][d, r, sl] = bufs[s][d, r, sl] * _SCALE

    def fire_write(s, g):
        pltpu.async_copy(
            bufs[s], out_hbm.at[pl.ds(base + g * _R0, _R0)], w_sems.at[s])

    def wait_write(s):
        pltpu.make_async_copy(
            bufs[s], out_hbm.at[pl.ds(base, _R0)], w_sems.at[s]).wait()

    fire_gathers(0, 0)
    fire_gathers(1, 1)

    @pl.loop(0, n_chunks // 2 - 1)
    def _steady(g2):
        c0 = g2 * 2
        for s in range(_NBUF):
            wait_gathers(s)
            scale(s)
            fire_write(s, c0 + s)
            wait_write(s)
            fire_gathers(s, c0 + s + 2)

    for s in range(_NBUF):
        wait_gathers(s)
        scale(s)
        fire_write(s, n_chunks - 2 + s)
    for s in range(_NBUF):
        wait_write(s)


def kernel(x, table):
    b0, b1 = x.shape
    idx = x.reshape(b0 * b1).astype(jnp.int32) * 2
    tab = jnp.pad(table, ((0, 0), (0, _D))).reshape(2 * table.shape[0], _D)

    info = plsc.get_sparse_core_info()
    nc, ns = info.num_cores, info.num_subcores
    nw = nc * ns
    w_rows = b0 // nw
    n_chunks = w_rows // _R0

    mesh = plsc.VectorSubcoreMesh(core_axis_name="c", subcore_axis_name="s")
    emb = pl.kernel(
        functools.partial(_emb_body, nc=nc, w_rows=w_rows, b1=b1,
                          n_chunks=n_chunks),
        out_type=jax.ShapeDtypeStruct((b0, b1, _D), jnp.float32),
        mesh=mesh,
        compiler_params=pltpu.CompilerParams(
            use_tc_tiling_on_sc=False, needs_layout_passes=True),
        scratch_types=[
            pltpu.VMEM((b0 * b1 // nw,), jnp.int32),
            pltpu.VMEM((_R0, b1, _D), jnp.float32),
            pltpu.VMEM((_R0, b1, _D), jnp.float32),
            pltpu.SemaphoreType.DMA((_NBUF,)),
            pltpu.SemaphoreType.DMA((_NBUF,)),
        ],
    )
    return emb(idx, tab)
